# baseline (speedup 1.0000x reference)
import functools

import jax
import jax.numpy as jnp
from jax import lax
from jax.experimental import pallas as pl
from jax.experimental.pallas import tpu as pltpu
from jax.experimental.pallas import tpu_sc as plsc

_LANES = 16


def kernel(group_scores, gid2verb):
    B, G = group_scores.shape
    (V,) = gid2verb.shape

    NW = 32
    RW = B // NW
    JB = 1024
    NCH = V // JB
    TAIL = V - NCH * JB

    mesh = plsc.VectorSubcoreMesh(core_axis_name="c", subcore_axis_name="s")

    @functools.partial(
        pl.kernel,
        mesh=mesh,
        out_type=jax.ShapeDtypeStruct((B, V), jnp.float32),
        compiler_params=pltpu.CompilerParams(
            use_tc_tiling_on_sc=False, needs_layout_passes=False
        ),
        scratch_types=[
            pltpu.VMEM((RW * G,), jnp.float32),
            pltpu.VMEM((JB,), jnp.int32),
            pltpu.VMEM((RW, JB), jnp.float32),
            pltpu.VMEM((TAIL,), jnp.int32),
            pltpu.VMEM((RW, TAIL), jnp.float32),
        ],
    )
    def gather_kernel(
        table_hbm, idx_hbm, out_hbm, table_v, idx_v, out_v, idxt_v, outt_v
    ):
        wid = lax.axis_index("s") * 2 + lax.axis_index("c")
        r0 = wid * RW
        pltpu.sync_copy(table_hbm.at[pl.ds(r0 * G, RW * G)], table_v)

        def gather_tile(iv, ov, width):
            def vec_body(jv, c):
                col = iv[pl.ds(jv * _LANES, _LANES)]
                for i in range(RW):
                    addr = col + jnp.int32(i * G)
                    ov[i, pl.ds(jv * _LANES, _LANES)] = plsc.load_gather(
                        table_v, [addr]
                    )
                return c

            lax.fori_loop(0, width // _LANES, vec_body, 0)

        def chunk_body(ci, carry):
            j0 = ci * JB
            pltpu.sync_copy(idx_hbm.at[pl.ds(j0, JB)], idx_v)
            gather_tile(idx_v, out_v, JB)
            pltpu.sync_copy(out_v, out_hbm.at[pl.ds(r0, RW), pl.ds(j0, JB)])
            return carry

        lax.fori_loop(0, NCH, chunk_body, 0)

        jt = NCH * JB
        pltpu.sync_copy(idx_hbm.at[pl.ds(jt, TAIL)], idxt_v)
        gather_tile(idxt_v, outt_v, TAIL)
        pltpu.sync_copy(outt_v, out_hbm.at[pl.ds(r0, RW), pl.ds(jt, TAIL)])

    return gather_kernel(group_scores.reshape(-1), gid2verb)


# speedup vs baseline: 1.7452x; 1.7452x over previous
import functools

import jax
import jax.numpy as jnp
from jax import lax
from jax.experimental import pallas as pl
from jax.experimental.pallas import tpu as pltpu
from jax.experimental.pallas import tpu_sc as plsc

_LANES = 16


def kernel(group_scores, gid2verb):
    B, G = group_scores.shape
    (V,) = gid2verb.shape

    NW = 32
    RW = B // NW
    JB = 1280
    NCH = V // JB
    TAIL = V - NCH * JB

    mesh = plsc.VectorSubcoreMesh(core_axis_name="c", subcore_axis_name="s")

    @functools.partial(
        pl.kernel,
        mesh=mesh,
        out_type=jax.ShapeDtypeStruct((B, V), jnp.float32),
        compiler_params=pltpu.CompilerParams(
            use_tc_tiling_on_sc=False, needs_layout_passes=False
        ),
        scratch_types=[
            pltpu.VMEM((RW * G,), jnp.float32),
            pltpu.VMEM((2, JB), jnp.int32),
            pltpu.VMEM((2, RW, JB), jnp.float32),
            pltpu.SemaphoreType.DMA,
            pltpu.SemaphoreType.DMA,
            pltpu.SemaphoreType.DMA,
            pltpu.SemaphoreType.DMA,
        ],
    )
    def gather_kernel(
        table_hbm, idx_hbm, out_hbm,
        table_v, idx_v, out_v, sem_i0, sem_i1, sem_o0, sem_o1
    ):
        sem_i = (sem_i0, sem_i1)
        sem_o = (sem_o0, sem_o1)
        wid = lax.axis_index("s") * 2 + lax.axis_index("c")
        r0 = wid * RW
        pltpu.sync_copy(table_hbm.at[pl.ds(r0 * G, RW * G)], table_v)

        def idx_copy(ci, b):
            return pltpu.make_async_copy(
                idx_hbm.at[pl.ds(ci * JB, JB)], idx_v.at[b], sem_i[b]
            )

        def out_copy(ci, b):
            return pltpu.make_async_copy(
                out_v.at[b],
                out_hbm.at[pl.ds(r0, RW), pl.ds(ci * JB, JB)],
                sem_o[b],
            )

        def gather_tile(iv, ov, width):
            @plsc.parallel_loop(0, width // _LANES, unroll=2)
            def vec_body(jv):
                base = jv * _LANES
                col = iv[pl.ds(base, _LANES)]
                vals = [
                    plsc.load_gather(table_v, [col + jnp.int32(i * G)])
                    for i in range(RW)
                ]
                for i in range(RW):
                    ov[i, pl.ds(base, _LANES)] = vals[i]

        idx_copy(0, 0).start()
        idx_copy(1, 1).start()

        def pair_body(g, carry):
            for b in (0, 1):
                ci = g + b
                idx_copy(ci, b).wait()

                @pl.when(g > 0)
                def _():
                    out_copy(ci - 2, b).wait()

                gather_tile(idx_v.at[b], out_v.at[b], JB)

                @pl.when(g < NCH - 2)
                def _():
                    idx_copy(ci + 2, b).start()

                out_copy(ci, b).start()
            return carry

        lax.fori_loop(0, NCH // 2, lambda h, c: pair_body(h * 2, c), 0)

        out_copy(NCH - 2, 0).wait()
        pltpu.sync_copy(
            idx_hbm.at[pl.ds(NCH * JB, TAIL)], idx_v.at[0, pl.ds(0, TAIL)]
        )
        gather_tile(idx_v.at[0], out_v.at[0], TAIL)
        tail_copy = pltpu.make_async_copy(
            out_v.at[0, :, pl.ds(0, TAIL)],
            out_hbm.at[pl.ds(r0, RW), pl.ds(NCH * JB, TAIL)],
            sem_o[0],
        )
        tail_copy.start()
        tail_copy.wait()

        out_copy(NCH - 1, 1).wait()

    return gather_kernel(group_scores.reshape(-1), gid2verb)


# speedup vs baseline: 3.3716x; 1.9319x over previous
import functools

import jax
import jax.numpy as jnp
from jax import lax
from jax.experimental import pallas as pl
from jax.experimental.pallas import tpu as pltpu
from jax.experimental.pallas import tpu_sc as plsc

_LANES = 16


def kernel(group_scores, gid2verb):
    B, G = group_scores.shape
    (V,) = gid2verb.shape

    NW = 32
    RW = B // NW
    JB = 1280
    NCH = V // JB
    TAIL = V - NCH * JB

    mesh = plsc.VectorSubcoreMesh(core_axis_name="c", subcore_axis_name="s")

    @functools.partial(
        pl.kernel,
        mesh=mesh,
        out_type=jax.ShapeDtypeStruct((B, V), jnp.float32),
        compiler_params=pltpu.CompilerParams(needs_layout_passes=False),
        scratch_types=[
            pltpu.VMEM((RW * G,), jnp.float32),
            pltpu.VMEM((JB,), jnp.int32),
            pltpu.VMEM((JB,), jnp.int32),
            pltpu.VMEM((RW, JB), jnp.float32),
            pltpu.VMEM((RW, JB), jnp.float32),
            pltpu.VMEM((TAIL,), jnp.int32),
            pltpu.VMEM((RW, TAIL), jnp.float32),
            pltpu.SemaphoreType.DMA,
            pltpu.SemaphoreType.DMA,
            pltpu.SemaphoreType.DMA,
            pltpu.SemaphoreType.DMA,
        ],
    )
    def gather_kernel(
        table_hbm, idx_hbm, out_hbm,
        table_v, idx_v0, idx_v1, out_v0, out_v1, idxt_v, outt_v,
        sem_i0, sem_i1, sem_o0, sem_o1
    ):
        idx_v = (idx_v0, idx_v1)
        out_v = (out_v0, out_v1)
        sem_i = (sem_i0, sem_i1)
        sem_o = (sem_o0, sem_o1)
        wid = lax.axis_index("s") * 2 + lax.axis_index("c")
        r0 = wid * RW
        pltpu.sync_copy(table_hbm.at[pl.ds(r0 * G, RW * G)], table_v)

        def idx_copy(ci, b):
            return pltpu.make_async_copy(
                idx_hbm.at[pl.ds(ci * JB, JB)], idx_v[b], sem_i[b]
            )

        def out_copy(ci, b):
            return pltpu.make_async_copy(
                out_v[b],
                out_hbm.at[pl.ds(r0, RW), pl.ds(ci * JB, JB)],
                sem_o[b],
            )

        def gather_tile(iv, ov, width):
            @plsc.parallel_loop(0, width // _LANES, unroll=2)
            def vec_body(jv):
                base = jv * _LANES
                col = iv[pl.ds(base, _LANES)]
                vals = [
                    plsc.load_gather(table_v, [col + jnp.int32(i * G)])
                    for i in range(RW)
                ]
                for i in range(RW):
                    ov[i, pl.ds(base, _LANES)] = vals[i]

        idx_copy(0, 0).start()
        idx_copy(1, 1).start()

        def pair_body(g, carry):
            for b in (0, 1):
                ci = g + b
                idx_copy(ci, b).wait()

                @pl.when(g > 0)
                def _():
                    out_copy(ci - 2, b).wait()

                gather_tile(idx_v[b], out_v[b], JB)

                @pl.when(g < NCH - 2)
                def _():
                    idx_copy(ci + 2, b).start()

                out_copy(ci, b).start()
            return carry

        lax.fori_loop(0, NCH // 2, lambda h, c: pair_body(h * 2, c), 0)

        pltpu.sync_copy(idx_hbm.at[pl.ds(NCH * JB, TAIL)], idxt_v)
        gather_tile(idxt_v, outt_v, TAIL)
        pltpu.sync_copy(
            outt_v, out_hbm.at[pl.ds(r0, RW), pl.ds(NCH * JB, TAIL)]
        )

        out_copy(NCH - 2, 0).wait()
        out_copy(NCH - 1, 1).wait()

    return gather_kernel(group_scores.reshape(-1), gid2verb)


# speedup vs baseline: 6.1272x; 1.8173x over previous
import functools

import jax
import jax.numpy as jnp
from jax import lax
from jax.experimental import pallas as pl
from jax.experimental.pallas import tpu as pltpu
from jax.experimental.pallas import tpu_sc as plsc


def kernel(group_scores, gid2verb):
    B, G = group_scores.shape
    (V,) = gid2verb.shape

    NW = 32
    JW = 3128
    C = 48
    NCH = JW // C
    TAIL = JW - NCH * C

    mesh = plsc.VectorSubcoreMesh(core_axis_name="c", subcore_axis_name="s")

    @functools.partial(
        pl.kernel,
        mesh=mesh,
        out_type=jax.ShapeDtypeStruct((V, B), jnp.float32),
        compiler_params=pltpu.CompilerParams(needs_layout_passes=False),
        scratch_types=[
            pltpu.VMEM((JW,), jnp.int32),
            pltpu.VMEM((C, B), jnp.float32),
            pltpu.VMEM((C, B), jnp.float32),
            pltpu.SemaphoreType.DMA,
            pltpu.SemaphoreType.DMA,
            pltpu.SemaphoreType.DMA,
            pltpu.SemaphoreType.DMA,
        ],
    )
    def gather_kernel(
        tab_hbm, idx_hbm, out_hbm, idx_v, rows0, rows1, sg0, sg1, sw0, sw1
    ):
        rows = (rows0, rows1)
        sem_g = (sg0, sg1)
        sem_w = (sw0, sw1)
        wid = lax.axis_index("s") * 2 + lax.axis_index("c")
        j0 = jnp.where(wid == NW - 1, V - JW, wid * JW)
        pltpu.sync_copy(idx_hbm.at[pl.ds(j0, JW)], idx_v)

        def do_chunk(ci, width, b):
            @pl.when(ci >= 2)
            def _():
                pltpu.make_async_copy(
                    rows[b].at[pl.ds(0, C)],
                    out_hbm.at[pl.ds(j0 + (ci - 2) * C, C)],
                    sem_w[b],
                ).wait()

            gather = pltpu.make_async_copy(
                tab_hbm.at[idx_v.at[pl.ds(ci * C, width)]],
                rows[b].at[pl.ds(0, width)],
                sem_g[b],
            )
            gather.start()
            gather.wait()
            pltpu.make_async_copy(
                rows[b].at[pl.ds(0, width)],
                out_hbm.at[pl.ds(j0 + ci * C, width)],
                sem_w[b],
            ).start()

        def pair_body(g, carry):
            do_chunk(g * 2, C, 0)
            do_chunk(g * 2 + 1, C, 1)
            return carry

        lax.fori_loop(0, NCH // 2, pair_body, 0)

        do_chunk(NCH - 1, C, 0)
        do_chunk(NCH, TAIL, 1)

        pltpu.make_async_copy(
            rows[0].at[pl.ds(0, C)],
            out_hbm.at[pl.ds(j0 + (NCH - 1) * C, C)],
            sem_w[0],
        ).wait()
        pltpu.make_async_copy(
            rows[1].at[pl.ds(0, TAIL)],
            out_hbm.at[pl.ds(j0 + NCH * C, TAIL)],
            sem_w[1],
        ).wait()

    out_t = gather_kernel(group_scores.T, gid2verb)
    return out_t.T


# speedup vs baseline: 6.1983x; 1.0116x over previous
import functools

import jax
import jax.numpy as jnp
from jax import lax
from jax.experimental import pallas as pl
from jax.experimental.pallas import tpu as pltpu
from jax.experimental.pallas import tpu_sc as plsc


def kernel(group_scores, gid2verb):
    B, G = group_scores.shape
    (V,) = gid2verb.shape

    NW = 32
    JW = 3128
    C = 24
    NCH = JW // C
    TAIL = JW - NCH * C
    LAST = NCH
    NB = 4

    mesh = plsc.VectorSubcoreMesh(core_axis_name="c", subcore_axis_name="s")

    @functools.partial(
        pl.kernel,
        mesh=mesh,
        out_type=jax.ShapeDtypeStruct((V, B), jnp.float32),
        compiler_params=pltpu.CompilerParams(needs_layout_passes=False),
        scratch_types=[
            pltpu.VMEM((JW,), jnp.int32),
            pltpu.VMEM((C, B), jnp.float32),
            pltpu.VMEM((C, B), jnp.float32),
            pltpu.VMEM((C, B), jnp.float32),
            pltpu.VMEM((C, B), jnp.float32),
            pltpu.SemaphoreType.DMA,
            pltpu.SemaphoreType.DMA,
            pltpu.SemaphoreType.DMA,
            pltpu.SemaphoreType.DMA,
            pltpu.SemaphoreType.DMA,
            pltpu.SemaphoreType.DMA,
            pltpu.SemaphoreType.DMA,
            pltpu.SemaphoreType.DMA,
        ],
    )
    def gather_kernel(
        tab_hbm, idx_hbm, out_hbm, idx_v,
        rows0, rows1, rows2, rows3,
        sg0, sg1, sg2, sg3, sw0, sw1, sw2, sw3,
    ):
        rows = (rows0, rows1, rows2, rows3)
        sem_g = (sg0, sg1, sg2, sg3)
        sem_w = (sw0, sw1, sw2, sw3)
        wid = lax.axis_index("s") * 2 + lax.axis_index("c")
        j0 = jnp.where(wid == NW - 1, V - JW, wid * JW)
        pltpu.sync_copy(idx_hbm.at[pl.ds(j0, JW)], idx_v)

        def g_copy(ci, b, width):
            return pltpu.make_async_copy(
                tab_hbm.at[idx_v.at[pl.ds(ci * C, width)]],
                rows[b].at[pl.ds(0, width)],
                sem_g[b],
            )

        def w_copy(ci, b, width):
            return pltpu.make_async_copy(
                rows[b].at[pl.ds(0, width)],
                out_hbm.at[pl.ds(j0 + ci * C, width)],
                sem_w[b],
            )

        def step(ci, b, width=C, pre_w=None, pre_g=None, pre_gw=C):
            g_copy(ci, b, width).wait()
            w_copy(ci, b, width).start()
            nb = (b + 2) % NB
            if pre_w is not None:
                w_copy(pre_w, nb, C).wait()
            if pre_g is not None:
                g_copy(pre_g, nb, pre_gw).start()

        g_copy(0, 0, C).start()
        g_copy(1, 1, C).start()
        step(0, 0, pre_g=2)
        step(1, 1, pre_g=3)
        step(2, 2, pre_w=0, pre_g=4)
        step(3, 3, pre_w=1, pre_g=5)

        def quad_body(h, carry):
            ci = h * 4
            for u in range(4):
                step(ci + u, u, pre_w=ci + u - 2, pre_g=ci + u + 2)
            return carry

        lax.fori_loop(1, NCH // 4, quad_body, 0)

        step(NCH - 2, 0, pre_w=NCH - 4, pre_g=LAST, pre_gw=TAIL)
        step(NCH - 1, 1)
        step(LAST, 2, width=TAIL)
        w_copy(NCH - 3, 3, C).wait()
        w_copy(NCH - 2, 0, C).wait()
        w_copy(NCH - 1, 1, C).wait()
        w_copy(LAST, 2, TAIL).wait()

    out_t = gather_kernel(group_scores.T, gid2verb)
    return out_t.T
